# baseline (speedup 1.0000x reference)
import math

import jax
import jax.numpy as jnp
import numpy as np
from jax.experimental import pallas as pl

D = 128
DFF = 512
E = 8
H = 4
T = 5
NSEG = 30
SEGLEN = 250
NFFT = 64
HOP = 32
NFREQ = 33
NFRAMES = 8
FLAT = NFREQ * NFRAMES
_ISQ2 = 1.0 / math.sqrt(2.0)
NTOK = 8 * NSEG
NPAD = 256
NREAL = NTOK + 1
PADLEN = SEGLEN + NFFT


def _make_dft_mats():
    x = np.arange(NFFT)
    q = np.arange(NFREQ)[None, :]
    ang = 2.0 * np.pi * x[:, None] * q / NFFT
    C = np.cos(ang)
    S = np.sin(ang)
    CF = np.zeros((SEGLEN, FLAT), np.float64)
    SF = np.zeros((SEGLEN, FLAT), np.float64)
    for fr in range(NFRAMES):
        for xi in range(NFFT):
            jp = fr * HOP + xi
            if jp < NFFT // 2:
                si = NFFT // 2 - jp
            elif jp < NFFT // 2 + SEGLEN:
                si = jp - NFFT // 2
            else:
                si = (SEGLEN - 2) - (jp - (NFFT // 2 + SEGLEN))
            CF[si, q[0] * NFRAMES + fr] += C[xi]
            SF[si, q[0] * NFRAMES + fr] += S[xi]
    return CF, SF


_CF64, _SF64 = _make_dft_mats()
_CF = _CF64.astype(np.float32)
_SF = _SF64.astype(np.float32)

_EXPAND = np.repeat(np.eye(E, dtype=np.float32), DFF, axis=1)


def _gelu(x):
    return 0.5 * x * (1.0 + jax.lax.erf(x * (1.0 / math.sqrt(2.0))))


def _ln(x, g, b, eps=1e-5):
    m = jnp.mean(x, axis=-1, keepdims=True)
    v = jnp.mean((x - m) ** 2, axis=-1, keepdims=True)
    return (x - m) * jax.lax.rsqrt(v + eps) * g + b


def _fused_kernel(xs_ref, tid_ref,
                  cb_ref, sb_ref, w2_ref, bp_ref, lng_ref, lnb_ref,
                  pos_ref, cls_ref,
                  n1g_ref, n1b_ref, n2g_ref, n2b_ref, nfg_ref, nfb_ref,
                  wq_ref, bq_ref, wk_ref, bk_ref, wv_ref, bv_ref,
                  wo_ref, bo_ref,
                  te_ref, wg_ref, bg_ref,
                  we1_ref, be1_ref, we2_ref, be2_ref, exp_ref,
                  wu1_ref, bu1_ref, wu2_ref, bu2_ref,
                  hw_ref, hb_ref,
                  gl_ref, tl_ref):
    dh = D // H
    f32 = jnp.float32

    xs = xs_ref[0]
    re = jnp.dot(xs, cb_ref[...], preferred_element_type=f32)
    im = jnp.dot(xs, sb_ref[...], preferred_element_type=f32)
    mag = jnp.sqrt(re * re + im * im)
    tok = jnp.dot(mag, w2_ref[...], preferred_element_type=f32) + bp_ref[...]
    tok = _gelu(_ln(tok, lng_ref[...], lnb_ref[...])) + pos_ref[0]

    src = jnp.concatenate(
        [cls_ref[...], tok, jnp.zeros((NPAD - NREAL, D), f32)], axis=0)

    s2 = _ln(src, n1g_ref[...], n1b_ref[...])
    q = jnp.dot(s2, wq_ref[...], preferred_element_type=f32) + bq_ref[...]
    k = jnp.dot(s2, wk_ref[...], preferred_element_type=f32) + bk_ref[...]
    v = jnp.dot(s2, wv_ref[...], preferred_element_type=f32) + bv_ref[...]

    col = jax.lax.broadcasted_iota(jnp.int32, (NPAD, NPAD), 1)
    key_mask = col < NREAL
    scale = 1.0 / math.sqrt(dh)
    heads = []
    for h in range(H):
        sl = slice(h * dh, (h + 1) * dh)
        sc = jnp.dot(q[:, sl], k[:, sl].T, preferred_element_type=f32) * scale
        sc = jnp.where(key_mask, sc, -1e30)
        m = jnp.max(sc, axis=-1, keepdims=True)
        p = jnp.exp(sc - m)
        p = p / jnp.sum(p, axis=-1, keepdims=True)
        heads.append(jnp.dot(p, v[:, sl], preferred_element_type=f32))
    ao = jnp.dot(jnp.concatenate(heads, axis=1), wo_ref[...],
                 preferred_element_type=f32) + bo_ref[...]
    src = src + ao

    s2 = _ln(src, n2g_ref[...], n2b_ref[...])
    oh = jnp.where(jax.lax.broadcasted_iota(jnp.int32, (1, E), 1) == tid_ref[0],
                   1.0, 0.0)
    tvec = jnp.dot(oh[:, :T], te_ref[...], preferred_element_type=f32)
    gl = (jnp.dot(s2, wg_ref[:D], preferred_element_type=f32)
          + jnp.dot(tvec, wg_ref[D:], preferred_element_type=f32)
          + bg_ref[...])
    gl_ref[0] = gl[:NREAL]

    eidx = jax.lax.broadcasted_iota(jnp.int32, (NPAD, E), 1)
    m1 = jnp.max(gl, axis=-1, keepdims=True)
    i1 = jnp.min(jnp.where(gl == m1, eidx, E), axis=-1, keepdims=True)
    oh1 = eidx == i1
    gl2 = jnp.where(oh1, -1e30, gl)
    m2 = jnp.max(gl2, axis=-1, keepdims=True)
    i2 = jnp.min(jnp.where(gl2 == m2, eidx, E), axis=-1, keepdims=True)
    sel = oh1 | (eidx == i2)
    es = jnp.where(sel, jnp.exp(gl - m1), 0.0)
    z = jnp.sum(es, axis=-1, keepdims=True)
    gates = es / z
    omega = 1.0 - jnp.max(gates, axis=-1, keepdims=True)

    t = jnp.dot(s2, we1_ref[...], preferred_element_type=f32) + be1_ref[...]
    u = t * jax.lax.erf(t) + t
    ge = jnp.dot(gates, exp_ref[...], preferred_element_type=f32)
    acc = (jnp.dot(u * ge, we2_ref[...], preferred_element_type=f32)
           + jnp.dot(gates, be2_ref[...], preferred_element_type=f32))

    tu = jnp.dot(s2, wu1_ref[...], preferred_element_type=f32) + bu1_ref[...]
    uu = tu * jax.lax.erf(tu) + tu
    univ = jnp.dot(uu, wu2_ref[...], preferred_element_type=f32) + bu2_ref[...]

    src = src + acc + omega * univ
    out = _ln(src, nfg_ref[...], nfb_ref[...])
    cls = out[0:1, :]

    a = jnp.concatenate(
        [jnp.dot(cls, hw_ref[t], preferred_element_type=f32)
         for t in range(T)], axis=0)
    tl = (jnp.dot(oh[:, :T], a, preferred_element_type=f32)
          + jnp.dot(oh[:, :T], hb_ref[...], preferred_element_type=f32))
    tl_ref[0] = tl


def kernel(x, task_ids, params):
    p = params
    B = x.shape[0]
    f32 = jnp.float32

    xs = x.reshape(B, NTOK, SEGLEN)
    row2 = lambda a: a.reshape(1, -1)
    tid = task_ids.astype(jnp.int32).reshape(B, 1, 1)

    const = lambda shape: pl.BlockSpec(shape, lambda b: tuple(0 for _ in shape))
    gl_out, tl_out = pl.pallas_call(
        _fused_kernel,
        grid=(B,),
        in_specs=[
            pl.BlockSpec((1, NTOK, SEGLEN), lambda b: (b, 0, 0)),
            pl.BlockSpec((1, 1, 1), lambda b: (b, 0, 0)),
            const((SEGLEN, FLAT)), const((SEGLEN, FLAT)),
            const((FLAT, D)), const((1, D)), const((1, D)), const((1, D)),
            const((1, NTOK, D)), const((1, D)),
            const((1, D)), const((1, D)), const((1, D)), const((1, D)),
            const((1, D)), const((1, D)),
            const((D, D)), const((1, D)), const((D, D)), const((1, D)),
            const((D, D)), const((1, D)), const((D, D)), const((1, D)),
            const((T, D)), const((2 * D, E)), const((1, E)),
            const((D, E * DFF)), const((1, E * DFF)),
            const((E * DFF, D)), const((E, D)), const((E, E * DFF)),
            const((D, DFF)), const((1, DFF)), const((DFF, D)), const((1, D)),
            const((T, D, 2)), const((T, 2)),
        ],
        out_specs=[
            pl.BlockSpec((1, NREAL, E), lambda b: (b, 0, 0)),
            pl.BlockSpec((1, 1, 2), lambda b: (b, 0, 0)),
        ],
        out_shape=[
            jax.ShapeDtypeStruct((B, NREAL, E), f32),
            jax.ShapeDtypeStruct((B, 1, 2), f32),
        ],
    )(xs, tid,
      jnp.asarray(_CF), jnp.asarray(_SF), p['W_proj'],
      row2(p['b_proj']), row2(p['ln_proj_g']), row2(p['ln_proj_b']),
      p['pos_embed'], p['cls_token'].reshape(1, D),
      row2(p['norm1_g']), row2(p['norm1_b']),
      row2(p['norm2_g']), row2(p['norm2_b']),
      row2(p['normf_g']), row2(p['normf_b']),
      p['Wq'], row2(p['bq']), p['Wk'], row2(p['bk']),
      p['Wv'], row2(p['bv']), p['Wo'], row2(p['bo']),
      p['task_embed'], p['Wg'], row2(p['bg']),
      p['We1'].transpose(1, 0, 2).reshape(D, E * DFF) * _ISQ2,
      p['be1'].reshape(1, E * DFF) * _ISQ2,
      p['We2'].reshape(E * DFF, D) * _ISQ2, p['be2'], jnp.asarray(_EXPAND),
      p['Wu1'] * _ISQ2, row2(p['bu1']) * _ISQ2, p['Wu2'] * _ISQ2,
      row2(p['bu2']),
      p['head_W'], p['head_b'])

    return tl_out.reshape(B, 2), gl_out


# speedup vs baseline: 1.3211x; 1.3211x over previous
import math

import jax
import jax.numpy as jnp
import numpy as np
from jax.experimental import pallas as pl

D = 128
DFF = 512
E = 8
H = 4
T = 5
NSEG = 30
SEGLEN = 250
NFFT = 64
HOP = 32
NFREQ = 33
NFRAMES = 8
FLAT = NFREQ * NFRAMES
NTOK = 8 * NSEG
NPAD = 256
NREAL = NTOK + 1
PADLEN = SEGLEN + NFFT
SP = 2


def _make_dft_mats():
    x = np.arange(NFFT)
    q = np.arange(NFREQ)[None, :]
    ang = 2.0 * np.pi * x[:, None] * q / NFFT
    C = np.cos(ang)
    S = np.sin(ang)
    CF = np.zeros((SEGLEN, FLAT), np.float64)
    SF = np.zeros((SEGLEN, FLAT), np.float64)
    for fr in range(NFRAMES):
        for xi in range(NFFT):
            jp = fr * HOP + xi
            if jp < NFFT // 2:
                si = NFFT // 2 - jp
            elif jp < NFFT // 2 + SEGLEN:
                si = jp - NFFT // 2
            else:
                si = (SEGLEN - 2) - (jp - (NFFT // 2 + SEGLEN))
            CF[si, q[0] * NFRAMES + fr] += C[xi]
            SF[si, q[0] * NFRAMES + fr] += S[xi]
    return CF, SF


_CF64, _SF64 = _make_dft_mats()
_CF = _CF64.astype(np.float32)
_SF = _SF64.astype(np.float32)


def _gelu(x):
    return 0.5 * x * (1.0 + jax.lax.erf(x * (1.0 / math.sqrt(2.0))))


def _ln(x, g, b, eps=1e-5):
    m = jnp.mean(x, axis=-1, keepdims=True)
    v = jnp.mean((x - m) ** 2, axis=-1, keepdims=True)
    return (x - m) * jax.lax.rsqrt(v + eps) * g + b


def _fused_kernel(xs_ref, tid_ref,
                  cb_ref, sb_ref, w2_ref, bp_ref, lng_ref, lnb_ref,
                  pos_ref, cls_ref,
                  n1g_ref, n1b_ref, n2g_ref, n2b_ref, nfg_ref, nfb_ref,
                  wq_ref, bq_ref, wk_ref, bk_ref, wv_ref, bv_ref,
                  wo_ref, bo_ref,
                  te_ref, wg_ref, bg_ref,
                  we1_ref, be1_ref, we2_ref, be2_ref,
                  wu1_ref, bu1_ref, wu2_ref, bu2_ref,
                  hw_ref, hb_ref,
                  gl_ref, tl_ref):
    dh = D // H
    f32 = jnp.float32
    NR = SP * NPAD

    xs = jnp.concatenate([xs_ref[s] for s in range(SP)], axis=0)
    re = jnp.dot(xs, cb_ref[...], preferred_element_type=f32)
    im = jnp.dot(xs, sb_ref[...], preferred_element_type=f32)
    mag = jnp.sqrt(re * re + im * im)
    tok = jnp.dot(mag, w2_ref[...], preferred_element_type=f32) + bp_ref[...]
    tok = _gelu(_ln(tok, lng_ref[...], lnb_ref[...]))

    parts = []
    for s in range(SP):
        parts += [cls_ref[...], tok[s * NTOK:(s + 1) * NTOK] + pos_ref[0],
                  jnp.zeros((NPAD - NREAL, D), f32)]
    src = jnp.concatenate(parts, axis=0)

    s2 = _ln(src, n1g_ref[...], n1b_ref[...])
    q = jnp.dot(s2, wq_ref[...], preferred_element_type=f32) + bq_ref[...]
    k = jnp.dot(s2, wk_ref[...], preferred_element_type=f32) + bk_ref[...]
    v = jnp.dot(s2, wv_ref[...], preferred_element_type=f32) + bv_ref[...]

    col = jax.lax.broadcasted_iota(jnp.int32, (NPAD, NPAD), 1)
    key_mask = col < NREAL
    scale = 1.0 / math.sqrt(dh)
    samp = []
    for s in range(SP):
        rs = slice(s * NPAD, (s + 1) * NPAD)
        heads = []
        for h in range(H):
            sl = slice(h * dh, (h + 1) * dh)
            sc = jnp.dot(q[rs, sl], k[rs, sl].T,
                         preferred_element_type=f32) * scale
            sc = jnp.where(key_mask, sc, -1e30)
            m = jnp.max(sc, axis=-1, keepdims=True)
            p = jnp.exp(sc - m)
            p = p / jnp.sum(p, axis=-1, keepdims=True)
            heads.append(jnp.dot(p, v[rs, sl], preferred_element_type=f32))
        samp.append(jnp.concatenate(heads, axis=1))
    ao = jnp.dot(jnp.concatenate(samp, axis=0), wo_ref[...],
                 preferred_element_type=f32) + bo_ref[...]
    src = src + ao

    s2 = _ln(src, n2g_ref[...], n2b_ref[...])
    oh = jnp.where(jax.lax.broadcasted_iota(jnp.int32, (SP, E), 1)
                   == tid_ref[:, 0], 1.0, 0.0)
    tvec = jnp.dot(oh[:, :T], te_ref[...], preferred_element_type=f32)
    tadd = jnp.dot(tvec, wg_ref[D:], preferred_element_type=f32)
    rowh = jax.lax.broadcasted_iota(jnp.int32, (NR, E), 0) // NPAD
    gadd = tadd[0:1]
    for s in range(1, SP):
        gadd = jnp.where(rowh == s, tadd[s:s + 1], gadd)
    gl = (jnp.dot(s2, wg_ref[:D], preferred_element_type=f32)
          + gadd + bg_ref[...])
    for s in range(SP):
        gl_ref[s] = gl[s * NPAD: s * NPAD + NREAL]

    eidx = jax.lax.broadcasted_iota(jnp.int32, (NR, E), 1)
    m1 = jnp.max(gl, axis=-1, keepdims=True)
    i1 = jnp.min(jnp.where(gl == m1, eidx, E), axis=-1, keepdims=True)
    oh1 = eidx == i1
    gl2 = jnp.where(oh1, -1e30, gl)
    m2 = jnp.max(gl2, axis=-1, keepdims=True)
    i2 = jnp.min(jnp.where(gl2 == m2, eidx, E), axis=-1, keepdims=True)
    sel = oh1 | (eidx == i2)
    es = jnp.where(sel, jnp.exp(gl - m1), 0.0)
    z = jnp.sum(es, axis=-1, keepdims=True)
    gates = es / z
    omega = 1.0 - jnp.max(gates, axis=-1, keepdims=True)

    acc = jnp.zeros((NR, D), f32)
    for e in range(E):
        he = _gelu(jnp.dot(s2, we1_ref[e], preferred_element_type=f32)
                   + be1_ref[e])
        eo = jnp.dot(he, we2_ref[e], preferred_element_type=f32) + be2_ref[e]
        acc = acc + gates[:, e:e + 1] * eo

    univ = jnp.dot(_gelu(jnp.dot(s2, wu1_ref[...], preferred_element_type=f32)
                         + bu1_ref[...]),
                   wu2_ref[...], preferred_element_type=f32) + bu2_ref[...]

    src = src + acc + omega * univ
    out = _ln(src, nfg_ref[...], nfb_ref[...])

    for s in range(SP):
        cls = out[s * NPAD: s * NPAD + 1, :]
        a = jnp.concatenate(
            [jnp.dot(cls, hw_ref[t], preferred_element_type=f32)
             for t in range(T)], axis=0)
        ohs = oh[s:s + 1, :T]
        tl_ref[s] = (jnp.dot(ohs, a, preferred_element_type=f32)
                     + jnp.dot(ohs, hb_ref[...], preferred_element_type=f32))


def kernel(x, task_ids, params):
    p = params
    B = x.shape[0]
    f32 = jnp.float32

    xs = x.reshape(B, NTOK, SEGLEN)
    row2 = lambda a: a.reshape(1, -1)
    tid = task_ids.astype(jnp.int32).reshape(B, 1, 1)

    const = lambda shape: pl.BlockSpec(shape, lambda b: tuple(0 for _ in shape))
    gl_out, tl_out = pl.pallas_call(
        _fused_kernel,
        grid=(B // SP,),
        in_specs=[
            pl.BlockSpec((SP, NTOK, SEGLEN), lambda b: (b, 0, 0)),
            pl.BlockSpec((SP, 1, 1), lambda b: (b, 0, 0)),
            const((SEGLEN, FLAT)), const((SEGLEN, FLAT)),
            const((FLAT, D)), const((1, D)), const((1, D)), const((1, D)),
            const((1, NTOK, D)), const((1, D)),
            const((1, D)), const((1, D)), const((1, D)), const((1, D)),
            const((1, D)), const((1, D)),
            const((D, D)), const((1, D)), const((D, D)), const((1, D)),
            const((D, D)), const((1, D)), const((D, D)), const((1, D)),
            const((T, D)), const((2 * D, E)), const((1, E)),
            const((E, D, DFF)), const((E, 1, DFF)),
            const((E, DFF, D)), const((E, 1, D)),
            const((D, DFF)), const((1, DFF)), const((DFF, D)), const((1, D)),
            const((T, D, 2)), const((T, 2)),
        ],
        out_specs=[
            pl.BlockSpec((SP, NREAL, E), lambda b: (b, 0, 0)),
            pl.BlockSpec((SP, 1, 2), lambda b: (b, 0, 0)),
        ],
        out_shape=[
            jax.ShapeDtypeStruct((B, NREAL, E), f32),
            jax.ShapeDtypeStruct((B, 1, 2), f32),
        ],
    )(xs, tid,
      jnp.asarray(_CF), jnp.asarray(_SF), p['W_proj'],
      row2(p['b_proj']), row2(p['ln_proj_g']), row2(p['ln_proj_b']),
      p['pos_embed'], p['cls_token'].reshape(1, D),
      row2(p['norm1_g']), row2(p['norm1_b']),
      row2(p['norm2_g']), row2(p['norm2_b']),
      row2(p['normf_g']), row2(p['normf_b']),
      p['Wq'], row2(p['bq']), p['Wk'], row2(p['bk']),
      p['Wv'], row2(p['bv']), p['Wo'], row2(p['bo']),
      p['task_embed'], p['Wg'], row2(p['bg']),
      p['We1'], p['be1'].reshape(E, 1, DFF),
      p['We2'], p['be2'].reshape(E, 1, D),
      p['Wu1'], row2(p['bu1']), p['Wu2'], row2(p['bu2']),
      p['head_W'], p['head_b'])

    return tl_out.reshape(B, 2), gl_out


# speedup vs baseline: 1.3745x; 1.0404x over previous
import math

import jax
import jax.numpy as jnp
import numpy as np
from jax.experimental import pallas as pl

D = 128
DFF = 512
E = 8
H = 4
T = 5
NSEG = 30
SEGLEN = 250
NFFT = 64
HOP = 32
NFREQ = 33
NFRAMES = 8
FLAT = NFREQ * NFRAMES
NTOK = 8 * NSEG
NPAD = 256
NREAL = NTOK + 1
PADLEN = SEGLEN + NFFT
SP = 2


def _make_dft_mats():
    x = np.arange(NFFT)
    q = np.arange(NFREQ)[None, :]
    ang = 2.0 * np.pi * x[:, None] * q / NFFT
    C = np.cos(ang)
    S = np.sin(ang)
    CF = np.zeros((SEGLEN, FLAT), np.float64)
    SF = np.zeros((SEGLEN, FLAT), np.float64)
    for fr in range(NFRAMES):
        for xi in range(NFFT):
            jp = fr * HOP + xi
            if jp < NFFT // 2:
                si = NFFT // 2 - jp
            elif jp < NFFT // 2 + SEGLEN:
                si = jp - NFFT // 2
            else:
                si = (SEGLEN - 2) - (jp - (NFFT // 2 + SEGLEN))
            CF[si, q[0] * NFRAMES + fr] += C[xi]
            SF[si, q[0] * NFRAMES + fr] += S[xi]
    return CF, SF


_CF64, _SF64 = _make_dft_mats()
_CF = _CF64.astype(np.float32)
_SF = _SF64.astype(np.float32)

_RPICK = np.zeros((T, 2 * T), np.float32)
_GPICK = np.zeros((2 * T, 2), np.float32)
for _t in range(T):
    for _o in range(2):
        _RPICK[_t, 2 * _t + _o] = 1.0
        _GPICK[2 * _t + _o, _o] = 1.0


def _gelu(x):
    return 0.5 * x * (1.0 + jax.lax.erf(x * (1.0 / math.sqrt(2.0))))


def _ln(x, g, b, eps=1e-5):
    m = jnp.mean(x, axis=-1, keepdims=True)
    v = jnp.mean((x - m) ** 2, axis=-1, keepdims=True)
    return (x - m) * jax.lax.rsqrt(v + eps) * g + b


def _fused_kernel(xs_ref, tid_ref,
                  cb_ref, sb_ref, w2_ref, bp_ref, lng_ref, lnb_ref,
                  pos_ref, cls_ref,
                  n1g_ref, n1b_ref, n2g_ref, n2b_ref, nfg_ref, nfb_ref,
                  wq_ref, bq_ref, wk_ref, bk_ref, wv_ref, bv_ref,
                  wo_ref, bo_ref,
                  te_ref, wg_ref, bg_ref,
                  we1_ref, be1_ref, we2_ref, be2_ref,
                  wu1_ref, bu1_ref, wu2_ref, bu2_ref,
                  hw_ref, rp_ref, gp_ref, hb_ref,
                  gl_ref, tl_ref):
    dh = D // H
    f32 = jnp.float32
    NR = SP * NPAD

    xs = jnp.concatenate([xs_ref[s] for s in range(SP)], axis=0)
    re = jnp.dot(xs, cb_ref[...], preferred_element_type=f32)
    im = jnp.dot(xs, sb_ref[...], preferred_element_type=f32)
    mag = jnp.sqrt(re * re + im * im)
    tok = jnp.dot(mag, w2_ref[...], preferred_element_type=f32) + bp_ref[...]
    tok = _gelu(_ln(tok, lng_ref[...], lnb_ref[...]))

    parts = []
    for s in range(SP):
        parts += [cls_ref[...], tok[s * NTOK:(s + 1) * NTOK] + pos_ref[0],
                  jnp.zeros((NPAD - NREAL, D), f32)]
    src = jnp.concatenate(parts, axis=0)

    s2 = _ln(src, n1g_ref[...], n1b_ref[...])
    q = jnp.dot(s2, wq_ref[...], preferred_element_type=f32) + bq_ref[...]
    k = jnp.dot(s2, wk_ref[...], preferred_element_type=f32) + bk_ref[...]
    v = jnp.dot(s2, wv_ref[...], preferred_element_type=f32) + bv_ref[...]

    col = jax.lax.broadcasted_iota(jnp.int32, (NPAD, NPAD), 1)
    key_mask = col < NREAL
    scale = 1.0 / math.sqrt(dh)
    samp = []
    for s in range(SP):
        rs = slice(s * NPAD, (s + 1) * NPAD)
        heads = []
        for h in range(H):
            sl = slice(h * dh, (h + 1) * dh)
            sc = jnp.dot(q[rs, sl], k[rs, sl].T,
                         preferred_element_type=f32) * scale
            sc = jnp.where(key_mask, sc, -1e30)
            m = jnp.max(sc, axis=-1, keepdims=True)
            p = jnp.exp(sc - m)
            p = p / jnp.sum(p, axis=-1, keepdims=True)
            heads.append(jnp.dot(p, v[rs, sl], preferred_element_type=f32))
        samp.append(jnp.concatenate(heads, axis=1))
    ao = jnp.dot(jnp.concatenate(samp, axis=0), wo_ref[...],
                 preferred_element_type=f32) + bo_ref[...]
    src = src + ao

    s2 = _ln(src, n2g_ref[...], n2b_ref[...])
    oh = jnp.where(jax.lax.broadcasted_iota(jnp.int32, (SP, E), 1)
                   == tid_ref[:, 0], 1.0, 0.0)
    tvec = jnp.dot(oh[:, :T], te_ref[...], preferred_element_type=f32)
    tadd = jnp.dot(tvec, wg_ref[D:], preferred_element_type=f32)
    rowh = jax.lax.broadcasted_iota(jnp.int32, (NR, E), 0) // NPAD
    gadd = tadd[0:1]
    for s in range(1, SP):
        gadd = jnp.where(rowh == s, tadd[s:s + 1], gadd)
    gl = (jnp.dot(s2, wg_ref[:D], preferred_element_type=f32)
          + gadd + bg_ref[...])
    for s in range(SP):
        gl_ref[s] = gl[s * NPAD: s * NPAD + NREAL]

    eidx = jax.lax.broadcasted_iota(jnp.int32, (NR, E), 1)
    m1 = jnp.max(gl, axis=-1, keepdims=True)
    i1 = jnp.min(jnp.where(gl == m1, eidx, E), axis=-1, keepdims=True)
    oh1 = eidx == i1
    gl2 = jnp.where(oh1, -1e30, gl)
    m2 = jnp.max(gl2, axis=-1, keepdims=True)
    i2 = jnp.min(jnp.where(gl2 == m2, eidx, E), axis=-1, keepdims=True)
    sel = oh1 | (eidx == i2)
    es = jnp.where(sel, jnp.exp(gl - m1), 0.0)
    z = jnp.sum(es, axis=-1, keepdims=True)
    gates = es / z
    omega = 1.0 - jnp.max(gates, axis=-1, keepdims=True)

    acc = jnp.zeros((NR, D), f32)
    for e in range(E):
        he = _gelu(jnp.dot(s2, we1_ref[e], preferred_element_type=f32)
                   + be1_ref[e])
        eo = jnp.dot(he, we2_ref[e], preferred_element_type=f32) + be2_ref[e]
        acc = acc + gates[:, e:e + 1] * eo

    univ = jnp.dot(_gelu(jnp.dot(s2, wu1_ref[...], preferred_element_type=f32)
                         + bu1_ref[...]),
                   wu2_ref[...], preferred_element_type=f32) + bu2_ref[...]

    src = src + acc + omega * univ
    cls = jnp.concatenate([src[s * NPAD: s * NPAD + 1, :] for s in range(SP)],
                          axis=0)
    cls = _ln(cls, nfg_ref[...], nfb_ref[...])

    a = jnp.dot(cls, hw_ref[...], preferred_element_type=f32)
    picked = a * jnp.dot(oh[:, :T], rp_ref[...], preferred_element_type=f32)
    tl = (jnp.dot(picked, gp_ref[...], preferred_element_type=f32)
          + jnp.dot(oh[:, :T], hb_ref[...], preferred_element_type=f32))
    for s in range(SP):
        tl_ref[s] = tl[s:s + 1]


def kernel(x, task_ids, params):
    p = params
    B = x.shape[0]
    f32 = jnp.float32

    xs = x.reshape(B, NTOK, SEGLEN)
    row2 = lambda a: a.reshape(1, -1)
    tid = task_ids.astype(jnp.int32).reshape(B, 1, 1)

    const = lambda shape: pl.BlockSpec(shape, lambda b: tuple(0 for _ in shape))
    gl_out, tl_out = pl.pallas_call(
        _fused_kernel,
        grid=(B // SP,),
        in_specs=[
            pl.BlockSpec((SP, NTOK, SEGLEN), lambda b: (b, 0, 0)),
            pl.BlockSpec((SP, 1, 1), lambda b: (b, 0, 0)),
            const((SEGLEN, FLAT)), const((SEGLEN, FLAT)),
            const((FLAT, D)), const((1, D)), const((1, D)), const((1, D)),
            const((1, NTOK, D)), const((1, D)),
            const((1, D)), const((1, D)), const((1, D)), const((1, D)),
            const((1, D)), const((1, D)),
            const((D, D)), const((1, D)), const((D, D)), const((1, D)),
            const((D, D)), const((1, D)), const((D, D)), const((1, D)),
            const((T, D)), const((2 * D, E)), const((1, E)),
            const((E, D, DFF)), const((E, 1, DFF)),
            const((E, DFF, D)), const((E, 1, D)),
            const((D, DFF)), const((1, DFF)), const((DFF, D)), const((1, D)),
            const((D, 2 * T)), const((T, 2 * T)), const((2 * T, 2)),
            const((T, 2)),
        ],
        out_specs=[
            pl.BlockSpec((SP, NREAL, E), lambda b: (b, 0, 0)),
            pl.BlockSpec((SP, 1, 2), lambda b: (b, 0, 0)),
        ],
        out_shape=[
            jax.ShapeDtypeStruct((B, NREAL, E), f32),
            jax.ShapeDtypeStruct((B, 1, 2), f32),
        ],
    )(xs, tid,
      jnp.asarray(_CF), jnp.asarray(_SF), p['W_proj'],
      row2(p['b_proj']), row2(p['ln_proj_g']), row2(p['ln_proj_b']),
      p['pos_embed'], p['cls_token'].reshape(1, D),
      row2(p['norm1_g']), row2(p['norm1_b']),
      row2(p['norm2_g']), row2(p['norm2_b']),
      row2(p['normf_g']), row2(p['normf_b']),
      p['Wq'], row2(p['bq']), p['Wk'], row2(p['bk']),
      p['Wv'], row2(p['bv']), p['Wo'], row2(p['bo']),
      p['task_embed'], p['Wg'], row2(p['bg']),
      p['We1'], p['be1'].reshape(E, 1, DFF),
      p['We2'], p['be2'].reshape(E, 1, D),
      p['Wu1'], row2(p['bu1']), p['Wu2'], row2(p['bu2']),
      p['head_W'].transpose(1, 0, 2).reshape(D, 2 * T),
      jnp.asarray(_RPICK), jnp.asarray(_GPICK), p['head_b'])

    return tl_out.reshape(B, 2), gl_out


# speedup vs baseline: 1.4980x; 1.0899x over previous
import math

import jax
import jax.numpy as jnp
import numpy as np
from jax.experimental import pallas as pl

D = 128
DFF = 512
E = 8
H = 4
T = 5
NSEG = 30
SEGLEN = 250
NFFT = 64
HOP = 32
NFREQ = 33
NFRAMES = 8
FLAT = NFREQ * NFRAMES
NTOK = 8 * NSEG
NPAD = 256
NREAL = NTOK + 1
PADLEN = SEGLEN + NFFT
SP = 2


def _make_dft_mats():
    x = np.arange(NFFT)
    q = np.arange(NFREQ)[None, :]
    ang = 2.0 * np.pi * x[:, None] * q / NFFT
    C = np.cos(ang)
    S = np.sin(ang)
    CF = np.zeros((SEGLEN, FLAT), np.float64)
    SF = np.zeros((SEGLEN, FLAT), np.float64)
    for fr in range(NFRAMES):
        for xi in range(NFFT):
            jp = fr * HOP + xi
            if jp < NFFT // 2:
                si = NFFT // 2 - jp
            elif jp < NFFT // 2 + SEGLEN:
                si = jp - NFFT // 2
            else:
                si = (SEGLEN - 2) - (jp - (NFFT // 2 + SEGLEN))
            CF[si, q[0] * NFRAMES + fr] += C[xi]
            SF[si, q[0] * NFRAMES + fr] += S[xi]
    return CF, SF


_CF64, _SF64 = _make_dft_mats()
_CF = _CF64.astype(np.float32)
_SF = _SF64.astype(np.float32)

_MONES = np.full((D, D), 1.0 / D, np.float32)

_RPICK = np.zeros((T, 2 * T), np.float32)
_GPICK = np.zeros((2 * T, 2), np.float32)
for _t in range(T):
    for _o in range(2):
        _RPICK[_t, 2 * _t + _o] = 1.0
        _GPICK[2 * _t + _o, _o] = 1.0


def _gelu(x):
    return 0.5 * x * (1.0 + jax.lax.erf(x * (1.0 / math.sqrt(2.0))))


def _ln(x, g, b, eps=1e-5):
    m = jnp.mean(x, axis=-1, keepdims=True)
    v = jnp.mean((x - m) ** 2, axis=-1, keepdims=True)
    return (x - m) * jax.lax.rsqrt(v + eps) * g + b


def _ln_mxu(x, g, b, mo, eps=1e-5):
    mb = jnp.dot(x, mo, preferred_element_type=jnp.float32)
    d = x - mb
    vb = jnp.dot(d * d, mo, preferred_element_type=jnp.float32)
    return d * jax.lax.rsqrt(vb + eps) * g + b


def _fused_kernel(xs_ref, tid_ref,
                  cb_ref, sb_ref, w2_ref, bp_ref, lng_ref, lnb_ref,
                  pos_ref, cls_ref,
                  n1g_ref, n1b_ref, n2g_ref, n2b_ref, nfg_ref, nfb_ref,
                  wq_ref, bq_ref, wk_ref, bk_ref, wv_ref, bv_ref,
                  wo_ref, bo_ref,
                  te_ref, wg_ref, bg_ref,
                  we1_ref, be1_ref, we2_ref, be2_ref,
                  wu1_ref, bu1_ref, wu2_ref, bu2_ref,
                  hw_ref, rp_ref, gp_ref, hb_ref, mo_ref,
                  gl_ref, tl_ref):
    dh = D // H
    f32 = jnp.float32
    NR = SP * NPAD

    xs = jnp.concatenate([xs_ref[s] for s in range(SP)], axis=0)
    re = jnp.dot(xs, cb_ref[...], preferred_element_type=f32)
    im = jnp.dot(xs, sb_ref[...], preferred_element_type=f32)
    mag = jnp.sqrt(re * re + im * im)
    tok = jnp.dot(mag, w2_ref[...], preferred_element_type=f32) + bp_ref[...]
    tok = _gelu(_ln_mxu(tok, lng_ref[...], lnb_ref[...], mo_ref[...]))

    parts = []
    for s in range(SP):
        parts += [cls_ref[...], tok[s * NTOK:(s + 1) * NTOK] + pos_ref[0],
                  jnp.zeros((NPAD - NREAL, D), f32)]
    src = jnp.concatenate(parts, axis=0)

    s2 = _ln_mxu(src, n1g_ref[...], n1b_ref[...], mo_ref[...])
    q = jnp.dot(s2, wq_ref[...], preferred_element_type=f32) + bq_ref[...]
    kt = (jnp.dot(wk_ref[...], s2.T, preferred_element_type=f32)
          + bk_ref[...].T)
    v = jnp.dot(s2, wv_ref[...], preferred_element_type=f32) + bv_ref[...]

    col = jax.lax.broadcasted_iota(jnp.int32, (NPAD, NPAD), 1)
    key_mask = col < NREAL
    scale = 1.0 / math.sqrt(dh)
    samp = []
    for s in range(SP):
        rs = slice(s * NPAD, (s + 1) * NPAD)
        heads = []
        for h in range(H):
            sl = slice(h * dh, (h + 1) * dh)
            sc = jnp.dot(q[rs, sl], kt[sl, rs],
                         preferred_element_type=f32) * scale
            sc = jnp.where(key_mask, sc, -1e30)
            m = jnp.max(sc, axis=-1, keepdims=True)
            p = jnp.exp(sc - m)
            p = p / jnp.sum(p, axis=-1, keepdims=True)
            heads.append(jnp.dot(p, v[rs, sl], preferred_element_type=f32))
        samp.append(jnp.concatenate(heads, axis=1))
    ao = jnp.dot(jnp.concatenate(samp, axis=0), wo_ref[...],
                 preferred_element_type=f32) + bo_ref[...]
    src = src + ao

    s2 = _ln_mxu(src, n2g_ref[...], n2b_ref[...], mo_ref[...])
    oh = jnp.where(jax.lax.broadcasted_iota(jnp.int32, (SP, E), 1)
                   == tid_ref[:, 0], 1.0, 0.0)
    tvec = jnp.dot(oh[:, :T], te_ref[...], preferred_element_type=f32)
    tadd = jnp.dot(tvec, wg_ref[D:], preferred_element_type=f32)
    rowh = jax.lax.broadcasted_iota(jnp.int32, (NR, E), 0) // NPAD
    gadd = tadd[0:1]
    for s in range(1, SP):
        gadd = jnp.where(rowh == s, tadd[s:s + 1], gadd)
    gl = (jnp.dot(s2, wg_ref[:D], preferred_element_type=f32)
          + gadd + bg_ref[...])
    for s in range(SP):
        gl_ref[s] = gl[s * NPAD: s * NPAD + NREAL]

    eidx = jax.lax.broadcasted_iota(jnp.int32, (NR, E), 1)
    m1 = jnp.max(gl, axis=-1, keepdims=True)
    i1 = jnp.min(jnp.where(gl == m1, eidx, E), axis=-1, keepdims=True)
    oh1 = eidx == i1
    gl2 = jnp.where(oh1, -1e30, gl)
    m2 = jnp.max(gl2, axis=-1, keepdims=True)
    i2 = jnp.min(jnp.where(gl2 == m2, eidx, E), axis=-1, keepdims=True)
    sel = oh1 | (eidx == i2)
    es = jnp.where(sel, jnp.exp(gl - m1), 0.0)
    z = jnp.sum(es, axis=-1, keepdims=True)
    gates = es / z
    omega = 1.0 - jnp.max(gates, axis=-1, keepdims=True)

    acc = jnp.zeros((NR, D), f32)
    for e in range(E):
        he = _gelu(jnp.dot(s2, we1_ref[e], preferred_element_type=f32)
                   + be1_ref[e])
        eo = jnp.dot(he, we2_ref[e], preferred_element_type=f32) + be2_ref[e]
        acc = acc + gates[:, e:e + 1] * eo

    univ = jnp.dot(_gelu(jnp.dot(s2, wu1_ref[...], preferred_element_type=f32)
                         + bu1_ref[...]),
                   wu2_ref[...], preferred_element_type=f32) + bu2_ref[...]

    src = src + acc + omega * univ
    cls = jnp.concatenate([src[s * NPAD: s * NPAD + 1, :] for s in range(SP)],
                          axis=0)
    cls = _ln(cls, nfg_ref[...], nfb_ref[...])

    a = jnp.dot(cls, hw_ref[...], preferred_element_type=f32)
    picked = a * jnp.dot(oh[:, :T], rp_ref[...], preferred_element_type=f32)
    tl = (jnp.dot(picked, gp_ref[...], preferred_element_type=f32)
          + jnp.dot(oh[:, :T], hb_ref[...], preferred_element_type=f32))
    for s in range(SP):
        tl_ref[s] = tl[s:s + 1]


def kernel(x, task_ids, params):
    p = params
    B = x.shape[0]
    f32 = jnp.float32

    xs = x.reshape(B, NTOK, SEGLEN)
    row2 = lambda a: a.reshape(1, -1)
    tid = task_ids.astype(jnp.int32).reshape(B, 1, 1)

    const = lambda shape: pl.BlockSpec(shape, lambda b: tuple(0 for _ in shape))
    gl_out, tl_out = pl.pallas_call(
        _fused_kernel,
        grid=(B // SP,),
        in_specs=[
            pl.BlockSpec((SP, NTOK, SEGLEN), lambda b: (b, 0, 0)),
            pl.BlockSpec((SP, 1, 1), lambda b: (b, 0, 0)),
            const((SEGLEN, FLAT)), const((SEGLEN, FLAT)),
            const((FLAT, D)), const((1, D)), const((1, D)), const((1, D)),
            const((1, NTOK, D)), const((1, D)),
            const((1, D)), const((1, D)), const((1, D)), const((1, D)),
            const((1, D)), const((1, D)),
            const((D, D)), const((1, D)), const((D, D)), const((1, D)),
            const((D, D)), const((1, D)), const((D, D)), const((1, D)),
            const((T, D)), const((2 * D, E)), const((1, E)),
            const((E, D, DFF)), const((E, 1, DFF)),
            const((E, DFF, D)), const((E, 1, D)),
            const((D, DFF)), const((1, DFF)), const((DFF, D)), const((1, D)),
            const((D, 2 * T)), const((T, 2 * T)), const((2 * T, 2)),
            const((T, 2)), const((D, D)),
        ],
        out_specs=[
            pl.BlockSpec((SP, NREAL, E), lambda b: (b, 0, 0)),
            pl.BlockSpec((SP, 1, 2), lambda b: (b, 0, 0)),
        ],
        out_shape=[
            jax.ShapeDtypeStruct((B, NREAL, E), f32),
            jax.ShapeDtypeStruct((B, 1, 2), f32),
        ],
    )(xs, tid,
      jnp.asarray(_CF), jnp.asarray(_SF), p['W_proj'],
      row2(p['b_proj']), row2(p['ln_proj_g']), row2(p['ln_proj_b']),
      p['pos_embed'], p['cls_token'].reshape(1, D),
      row2(p['norm1_g']), row2(p['norm1_b']),
      row2(p['norm2_g']), row2(p['norm2_b']),
      row2(p['normf_g']), row2(p['normf_b']),
      p['Wq'], row2(p['bq']), p['Wk'].T, row2(p['bk']),
      p['Wv'], row2(p['bv']), p['Wo'], row2(p['bo']),
      p['task_embed'], p['Wg'], row2(p['bg']),
      p['We1'], p['be1'].reshape(E, 1, DFF),
      p['We2'], p['be2'].reshape(E, 1, D),
      p['Wu1'], row2(p['bu1']), p['Wu2'], row2(p['bu2']),
      p['head_W'].transpose(1, 0, 2).reshape(D, 2 * T),
      jnp.asarray(_RPICK), jnp.asarray(_GPICK), p['head_b'],
      jnp.asarray(_MONES))

    return tl_out.reshape(B, 2), gl_out


# speedup vs baseline: 1.5998x; 1.0679x over previous
import math

import jax
import jax.numpy as jnp
import numpy as np
from jax.experimental import pallas as pl

D = 128
DFF = 512
E = 8
H = 4
T = 5
NSEG = 30
SEGLEN = 250
NFFT = 64
HOP = 32
NFREQ = 33
NFRAMES = 8
FLAT = NFREQ * NFRAMES
NTOK = 8 * NSEG
NPAD = 256
NREAL = NTOK + 1
PADLEN = SEGLEN + NFFT
SP = 2


def _make_dft_mats():
    x = np.arange(NFFT)
    q = np.arange(NFREQ)[None, :]
    ang = 2.0 * np.pi * x[:, None] * q / NFFT
    C = np.cos(ang)
    S = np.sin(ang)
    CF = np.zeros((SEGLEN, FLAT), np.float64)
    SF = np.zeros((SEGLEN, FLAT), np.float64)
    for fr in range(NFRAMES):
        for xi in range(NFFT):
            jp = fr * HOP + xi
            if jp < NFFT // 2:
                si = NFFT // 2 - jp
            elif jp < NFFT // 2 + SEGLEN:
                si = jp - NFFT // 2
            else:
                si = (SEGLEN - 2) - (jp - (NFFT // 2 + SEGLEN))
            CF[si, q[0] * NFRAMES + fr] += C[xi]
            SF[si, q[0] * NFRAMES + fr] += S[xi]
    return CF, SF


_CF64, _SF64 = _make_dft_mats()
_CF = _CF64.astype(np.float32)
_SF = _SF64.astype(np.float32)

_MONES = np.full((D, D), 1.0 / D, np.float32)

_RPICK = np.zeros((T, 2 * T), np.float32)
_GPICK = np.zeros((2 * T, 2), np.float32)
for _t in range(T):
    for _o in range(2):
        _RPICK[_t, 2 * _t + _o] = 1.0
        _GPICK[2 * _t + _o, _o] = 1.0


def _gelu(x):
    return 0.5 * x * (1.0 + jax.lax.erf(x * (1.0 / math.sqrt(2.0))))


def _ln(x, g, b, eps=1e-5):
    m = jnp.mean(x, axis=-1, keepdims=True)
    v = jnp.mean((x - m) ** 2, axis=-1, keepdims=True)
    return (x - m) * jax.lax.rsqrt(v + eps) * g + b


def _ln_mxu(x, g, b, mo, eps=1e-5):
    mb = jnp.dot(x, mo, preferred_element_type=jnp.float32)
    d = x - mb
    vb = jnp.dot(d * d, mo, preferred_element_type=jnp.float32)
    return d * jax.lax.rsqrt(vb + eps) * g + b


def _fused_kernel(xs_ref, tid_ref,
                  cb_ref, sb_ref, w2_ref, bp_ref, lng_ref, lnb_ref,
                  pos_ref, cls_ref,
                  n1g_ref, n1b_ref, n2g_ref, n2b_ref, nfg_ref, nfb_ref,
                  wq_ref, bq_ref, wk_ref, bk_ref, wv_ref, bv_ref,
                  wo_ref, bo_ref,
                  te_ref, wg_ref, bg_ref,
                  we1_ref, be1_ref, we2_ref, be2_ref,
                  wu1_ref, bu1_ref, wu2_ref, bu2_ref,
                  hw_ref, rp_ref, gp_ref, hb_ref, mo_ref,
                  gl_ref, tl_ref):
    dh = D // H
    f32 = jnp.float32
    NR = SP * NPAD

    xs = jnp.concatenate([xs_ref[s] for s in range(SP)], axis=0)
    re = jnp.dot(xs, cb_ref[...], preferred_element_type=f32)
    im = jnp.dot(xs, sb_ref[...], preferred_element_type=f32)
    mag = jnp.sqrt(re * re + im * im)
    tok = jnp.dot(mag, w2_ref[...], preferred_element_type=f32) + bp_ref[...]
    tok = _gelu(_ln_mxu(tok, lng_ref[...], lnb_ref[...], mo_ref[...]))

    parts = []
    for s in range(SP):
        parts += [cls_ref[...], tok[s * NTOK:(s + 1) * NTOK] + pos_ref[0],
                  jnp.zeros((NPAD - NREAL, D), f32)]
    src = jnp.concatenate(parts, axis=0)

    s2 = _ln_mxu(src, n1g_ref[...], n1b_ref[...], mo_ref[...])
    q = jnp.dot(s2, wq_ref[...], preferred_element_type=f32) + bq_ref[...]
    kt = (jnp.dot(wk_ref[...], s2.T, preferred_element_type=f32)
          + bk_ref[...].T)
    v = jnp.dot(s2, wv_ref[...], preferred_element_type=f32) + bv_ref[...]

    col = jax.lax.broadcasted_iota(jnp.int32, (NPAD, NPAD), 1)
    key_mask = col < NREAL
    scale = 1.0 / math.sqrt(dh)
    samp = []
    for s in range(SP):
        rs = slice(s * NPAD, (s + 1) * NPAD)
        heads = []
        for h in range(H):
            sl = slice(h * dh, (h + 1) * dh)
            sc = jnp.dot(q[rs, sl], kt[sl, rs],
                         preferred_element_type=f32) * scale
            sc = jnp.where(key_mask, sc, -1e30)
            m = jnp.max(sc, axis=-1, keepdims=True)
            p = jnp.exp(sc - m)
            vv = jnp.concatenate([v[rs, sl], jnp.ones((NPAD, 8), f32)], axis=1)
            r = jnp.dot(p, vv, preferred_element_type=f32)
            heads.append(r[:, :dh] / r[:, dh:dh + 1])
        samp.append(jnp.concatenate(heads, axis=1))
    ao = jnp.dot(jnp.concatenate(samp, axis=0), wo_ref[...],
                 preferred_element_type=f32) + bo_ref[...]
    src = src + ao

    s2 = _ln_mxu(src, n2g_ref[...], n2b_ref[...], mo_ref[...])
    oh = jnp.where(jax.lax.broadcasted_iota(jnp.int32, (SP, E), 1)
                   == tid_ref[:, 0], 1.0, 0.0)
    tvec = jnp.dot(oh[:, :T], te_ref[...], preferred_element_type=f32)
    tadd = jnp.dot(tvec, wg_ref[D:], preferred_element_type=f32)
    rowh = jax.lax.broadcasted_iota(jnp.int32, (NR, E), 0) // NPAD
    gadd = tadd[0:1]
    for s in range(1, SP):
        gadd = jnp.where(rowh == s, tadd[s:s + 1], gadd)
    gl = (jnp.dot(s2, wg_ref[:D], preferred_element_type=f32)
          + gadd + bg_ref[...])
    for s in range(SP):
        gl_ref[s] = gl[s * NPAD: s * NPAD + NREAL]

    eidx = jax.lax.broadcasted_iota(jnp.int32, (NR, E), 1)
    m1 = jnp.max(gl, axis=-1, keepdims=True)
    i1 = jnp.min(jnp.where(gl == m1, eidx, E), axis=-1, keepdims=True)
    oh1 = eidx == i1
    gl2 = jnp.where(oh1, -1e30, gl)
    m2 = jnp.max(gl2, axis=-1, keepdims=True)
    i2 = jnp.min(jnp.where(gl2 == m2, eidx, E), axis=-1, keepdims=True)
    sel = oh1 | (eidx == i2)
    es = jnp.where(sel, jnp.exp(gl - m1), 0.0)
    z = jnp.sum(es, axis=-1, keepdims=True)
    gates = es / z
    omega = 1.0 - jnp.max(gates, axis=-1, keepdims=True)

    acc = jnp.zeros((NR, D), f32)
    for e in range(E):
        he = _gelu(jnp.dot(s2, we1_ref[e], preferred_element_type=f32)
                   + be1_ref[e])
        eo = jnp.dot(he, we2_ref[e], preferred_element_type=f32) + be2_ref[e]
        acc = acc + gates[:, e:e + 1] * eo

    univ = jnp.dot(_gelu(jnp.dot(s2, wu1_ref[...], preferred_element_type=f32)
                         + bu1_ref[...]),
                   wu2_ref[...], preferred_element_type=f32) + bu2_ref[...]

    src = src + acc + omega * univ
    cls = jnp.concatenate([src[s * NPAD: s * NPAD + 1, :] for s in range(SP)],
                          axis=0)
    cls = _ln(cls, nfg_ref[...], nfb_ref[...])

    a = jnp.dot(cls, hw_ref[...], preferred_element_type=f32)
    picked = a * jnp.dot(oh[:, :T], rp_ref[...], preferred_element_type=f32)
    tl = (jnp.dot(picked, gp_ref[...], preferred_element_type=f32)
          + jnp.dot(oh[:, :T], hb_ref[...], preferred_element_type=f32))
    for s in range(SP):
        tl_ref[s] = tl[s:s + 1]


def kernel(x, task_ids, params):
    p = params
    B = x.shape[0]
    f32 = jnp.float32

    xs = x.reshape(B, NTOK, SEGLEN)
    row2 = lambda a: a.reshape(1, -1)
    tid = task_ids.astype(jnp.int32).reshape(B, 1, 1)

    const = lambda shape: pl.BlockSpec(shape, lambda b: tuple(0 for _ in shape))
    gl_out, tl_out = pl.pallas_call(
        _fused_kernel,
        grid=(B // SP,),
        in_specs=[
            pl.BlockSpec((SP, NTOK, SEGLEN), lambda b: (b, 0, 0)),
            pl.BlockSpec((SP, 1, 1), lambda b: (b, 0, 0)),
            const((SEGLEN, FLAT)), const((SEGLEN, FLAT)),
            const((FLAT, D)), const((1, D)), const((1, D)), const((1, D)),
            const((1, NTOK, D)), const((1, D)),
            const((1, D)), const((1, D)), const((1, D)), const((1, D)),
            const((1, D)), const((1, D)),
            const((D, D)), const((1, D)), const((D, D)), const((1, D)),
            const((D, D)), const((1, D)), const((D, D)), const((1, D)),
            const((T, D)), const((2 * D, E)), const((1, E)),
            const((E, D, DFF)), const((E, 1, DFF)),
            const((E, DFF, D)), const((E, 1, D)),
            const((D, DFF)), const((1, DFF)), const((DFF, D)), const((1, D)),
            const((D, 2 * T)), const((T, 2 * T)), const((2 * T, 2)),
            const((T, 2)), const((D, D)),
        ],
        out_specs=[
            pl.BlockSpec((SP, NREAL, E), lambda b: (b, 0, 0)),
            pl.BlockSpec((SP, 1, 2), lambda b: (b, 0, 0)),
        ],
        out_shape=[
            jax.ShapeDtypeStruct((B, NREAL, E), f32),
            jax.ShapeDtypeStruct((B, 1, 2), f32),
        ],
    )(xs, tid,
      jnp.asarray(_CF), jnp.asarray(_SF), p['W_proj'],
      row2(p['b_proj']), row2(p['ln_proj_g']), row2(p['ln_proj_b']),
      p['pos_embed'], p['cls_token'].reshape(1, D),
      row2(p['norm1_g']), row2(p['norm1_b']),
      row2(p['norm2_g']), row2(p['norm2_b']),
      row2(p['normf_g']), row2(p['normf_b']),
      p['Wq'], row2(p['bq']), p['Wk'].T, row2(p['bk']),
      p['Wv'], row2(p['bv']), p['Wo'], row2(p['bo']),
      p['task_embed'], p['Wg'], row2(p['bg']),
      p['We1'], p['be1'].reshape(E, 1, DFF),
      p['We2'], p['be2'].reshape(E, 1, D),
      p['Wu1'], row2(p['bu1']), p['Wu2'], row2(p['bu2']),
      p['head_W'].transpose(1, 0, 2).reshape(D, 2 * T),
      jnp.asarray(_RPICK), jnp.asarray(_GPICK), p['head_b'],
      jnp.asarray(_MONES))

    return tl_out.reshape(B, 2), gl_out
